# Initial kernel scaffold; baseline (speedup 1.0000x reference)
import jax
import jax.numpy as jnp
from jax.experimental import pallas as pl


def kernel(x, edge_index, batch, params):
    raise NotImplementedError("write your pallas kernel here")


# speedup vs baseline: 6.3890x; 6.3890x over previous
import functools

import jax
import jax.numpy as jnp
from jax import lax
from jax.experimental import pallas as pl
from jax.experimental.pallas import tpu as pltpu
from jax.experimental.pallas import tpu_sc as plsc

N = 10000
E = 320000
D = 128
H = 128
G = 64
F = 8

NC = 2
NS = 16
NW = NC * NS
EPW = E // NW
W = 125
NWIN = EPW // W
NP = 10240
RPS = NP // NS

_sc_mesh = plsc.VectorSubcoreMesh(core_axis_name="c", subcore_axis_name="s")


@functools.partial(
    pl.kernel,
    out_type=jax.ShapeDtypeStruct((NC, NP, D), jnp.float32),
    mesh=_sc_mesh,
    scratch_types=[
        pltpu.VMEM((NWIN, W), jnp.int32),
        pltpu.VMEM((NWIN, W), jnp.int32),
        pltpu.VMEM((W, D), jnp.float32),
        pltpu.VMEM_SHARED((NP, D), jnp.float32),
        pltpu.SemaphoreType.DMA,
    ],
)
def _sc_agg(h_hbm, src_hbm, dst_hbm, zeros_hbm, out_hbm,
            src_v, dst_v, rows_v, agg_sh, sem):
    c = lax.axis_index("c")
    s = lax.axis_index("s")
    wid = s * NC + c
    pltpu.sync_copy(zeros_hbm, agg_sh.at[pl.ds(s * RPS, RPS)])
    pltpu.sync_copy(src_hbm.at[wid], src_v)
    pltpu.sync_copy(dst_hbm.at[wid], dst_v)
    plsc.subcore_barrier()

    @pl.loop(0, NWIN)
    def _(w):
        pltpu.async_copy(h_hbm.at[src_v.at[w]], rows_v, sem).wait()
        pltpu.sync_copy(rows_v, agg_sh.at[dst_v.at[w]], add=True)

    plsc.subcore_barrier()
    pltpu.sync_copy(agg_sh.at[pl.ds(s * RPS, RPS)],
                    out_hbm.at[c, pl.ds(s * RPS, RPS)])


def _tc_layer_body(h_ref, a0_ref, a1_ref, w1_ref, b1_ref, g1_ref, bt1_ref,
                   w2_ref, b2_ref, batch_ref, h_out_ref, pooled_ref):
    hin = h_ref[...] + a0_ref[...] + a1_ref[...]
    z = jnp.dot(hin, w1_ref[...], preferred_element_type=jnp.float32,
                precision=lax.Precision.HIGHEST) + b1_ref[...]
    mean = jnp.mean(z, axis=0, keepdims=True)
    zc = z - mean
    var = jnp.mean(zc * zc, axis=0, keepdims=True)
    zn = zc * lax.rsqrt(var + 1e-5) * g1_ref[...] + bt1_ref[...]
    zn = jnp.maximum(zn, 0.0)
    h2 = jnp.dot(zn, w2_ref[...], preferred_element_type=jnp.float32,
                 precision=lax.Precision.HIGHEST) + b2_ref[...]
    h2 = jnp.maximum(h2, 0.0)
    h_out_ref[...] = h2
    gids = lax.broadcasted_iota(jnp.int32, (G, N), 0)
    mask = (batch_ref[...] == gids).astype(jnp.float32)
    psum = jnp.dot(mask, h2, preferred_element_type=jnp.float32,
                   precision=lax.Precision.HIGHEST)
    cnt = jnp.maximum(jnp.sum(mask, axis=1, keepdims=True), 1.0)
    pooled_ref[...] = psum / cnt


_tc_layer = pl.pallas_call(
    _tc_layer_body,
    out_shape=(
        jax.ShapeDtypeStruct((N, H), jnp.float32),
        jax.ShapeDtypeStruct((G, H), jnp.float32),
    ),
)


def _tc_head_body(hcat_ref, w1_ref, b1_ref, w2_ref, b2_ref, out_ref):
    h = jnp.dot(hcat_ref[...], w1_ref[...], preferred_element_type=jnp.float32,
                precision=lax.Precision.HIGHEST) + b1_ref[...]
    h = jnp.maximum(h, 0.0)
    out_ref[...] = jnp.dot(h, w2_ref[...], preferred_element_type=jnp.float32,
                           precision=lax.Precision.HIGHEST) + b2_ref[...]


_tc_head = pl.pallas_call(
    _tc_head_body,
    out_shape=jax.ShapeDtypeStruct((G, F), jnp.float32),
)


def kernel(x, edge_index, batch, params):
    src = edge_index[0].reshape(NW, NWIN, W)
    dst = edge_index[1].reshape(NW, NWIN, W)
    zeros = jnp.zeros((RPS, D), jnp.float32)
    batch2 = batch.reshape(1, N)
    h = x
    pooled = []
    for name in ("conv1", "conv2", "conv3", "conv4", "conv5"):
        p = params[name]
        agg = _sc_agg(h, src, dst, zeros)
        h, pool = _tc_layer(
            h, agg[0, :N], agg[1, :N],
            p["W1"], p["b1"].reshape(1, H), p["g1"].reshape(1, H),
            p["beta1"].reshape(1, H), p["W2"], p["b2"].reshape(1, H),
            batch2)
        pooled.append(pool)
    hcat = jnp.concatenate(pooled, axis=1)
    out = _tc_head(hcat, params["lin1_W"], params["lin1_b"].reshape(1, 5 * H),
                   params["lin2_W"], params["lin2_b"].reshape(1, F))
    return out.reshape(-1)


# speedup vs baseline: 9.2999x; 1.4556x over previous
import functools

import jax
import jax.numpy as jnp
from jax import lax
from jax.experimental import pallas as pl
from jax.experimental.pallas import tpu as pltpu
from jax.experimental.pallas import tpu_sc as plsc

N = 10000
E = 320000
D = 128
H = 128
G = 64
F = 8

NC = 2
NS = 16
NW = NC * NS
EPW = E // NW
W = 125
NWIN = EPW // W
NP = 10240
RPS = NP // NS
CW = 8
NCH = NWIN // CW

_sc_mesh = plsc.VectorSubcoreMesh(core_axis_name="c", subcore_axis_name="s")


@functools.partial(
    pl.kernel,
    out_type=jax.ShapeDtypeStruct((NC, NP, D), jnp.float32),
    mesh=_sc_mesh,
    scratch_types=[
        pltpu.VMEM((2, CW, W), jnp.int32),
        pltpu.VMEM((2, CW, W), jnp.int32),
        pltpu.VMEM((2, W, D), jnp.float32),
        pltpu.VMEM_SHARED((NP, D), jnp.float32),
        pltpu.SemaphoreType.DMA,
        pltpu.SemaphoreType.DMA,
        pltpu.SemaphoreType.DMA,
    ],
)
def _sc_agg(h_hbm, src_hbm, dst_hbm, zeros_hbm, out_hbm,
            src_v, dst_v, rows_v, agg_sh, semi, sem0, sem1):
    c = lax.axis_index("c")
    s = lax.axis_index("s")
    wid = s * NC + c
    sems = (sem0, sem1)
    pltpu.sync_copy(zeros_hbm, agg_sh.at[pl.ds(s * RPS, RPS)])
    pltpu.sync_copy(src_hbm.at[wid, pl.ds(0, CW)], src_v.at[0])
    pltpu.sync_copy(dst_hbm.at[wid, pl.ds(0, CW)], dst_v.at[0])
    plsc.subcore_barrier()
    pltpu.async_copy(h_hbm.at[src_v.at[0, 0]], rows_v.at[0], sem0)

    @pl.loop(0, NCH)
    def _(ch):
        slot = lax.rem(ch, 2)
        nslot = 1 - slot

        @pl.when(ch + 1 < NCH)
        def _():
            pltpu.async_copy(src_hbm.at[wid, pl.ds((ch + 1) * CW, CW)],
                             src_v.at[nslot], semi)
            pltpu.async_copy(dst_hbm.at[wid, pl.ds((ch + 1) * CW, CW)],
                             dst_v.at[nslot], semi)

        for k in range(CW):
            if k + 1 < CW:
                pltpu.async_copy(h_hbm.at[src_v.at[slot, k + 1]],
                                 rows_v.at[(k + 1) % 2], sems[(k + 1) % 2])
            else:
                @pl.when(ch + 1 < NCH)
                def _():
                    pltpu.make_async_copy(
                        src_hbm.at[wid, pl.ds(0, CW)], src_v.at[nslot],
                        semi).wait()
                    pltpu.make_async_copy(
                        dst_hbm.at[wid, pl.ds(0, CW)], dst_v.at[nslot],
                        semi).wait()
                    pltpu.async_copy(h_hbm.at[src_v.at[nslot, 0]],
                                     rows_v.at[0], sem0)
            pltpu.make_async_copy(h_hbm.at[src_v.at[slot, k]],
                                  rows_v.at[k % 2], sems[k % 2]).wait()
            pltpu.sync_copy(rows_v.at[k % 2], agg_sh.at[dst_v.at[slot, k]],
                            add=True)

    plsc.subcore_barrier()
    pltpu.sync_copy(agg_sh.at[pl.ds(s * RPS, RPS)],
                    out_hbm.at[c, pl.ds(s * RPS, RPS)])


def _tc_layer_body(h_ref, a0_ref, a1_ref, w1_ref, b1_ref, g1_ref, bt1_ref,
                   w2_ref, b2_ref, batch_ref, h_out_ref, pooled_ref):
    hin = h_ref[...] + a0_ref[...] + a1_ref[...]
    z = jnp.dot(hin, w1_ref[...], preferred_element_type=jnp.float32,
                precision=lax.Precision.HIGHEST) + b1_ref[...]
    mean = jnp.mean(z, axis=0, keepdims=True)
    zc = z - mean
    var = jnp.mean(zc * zc, axis=0, keepdims=True)
    zn = zc * lax.rsqrt(var + 1e-5) * g1_ref[...] + bt1_ref[...]
    zn = jnp.maximum(zn, 0.0)
    h2 = jnp.dot(zn, w2_ref[...], preferred_element_type=jnp.float32,
                 precision=lax.Precision.HIGHEST) + b2_ref[...]
    h2 = jnp.maximum(h2, 0.0)
    h_out_ref[...] = h2
    gids = lax.broadcasted_iota(jnp.int32, (G, N), 0)
    mask = (batch_ref[...] == gids).astype(jnp.float32)
    psum = jnp.dot(mask, h2, preferred_element_type=jnp.float32,
                   precision=lax.Precision.HIGHEST)
    cnt = jnp.maximum(jnp.sum(mask, axis=1, keepdims=True), 1.0)
    pooled_ref[...] = psum / cnt


_tc_layer = pl.pallas_call(
    _tc_layer_body,
    out_shape=(
        jax.ShapeDtypeStruct((N, H), jnp.float32),
        jax.ShapeDtypeStruct((G, H), jnp.float32),
    ),
)


def _tc_head_body(hcat_ref, w1_ref, b1_ref, w2_ref, b2_ref, out_ref):
    h = jnp.dot(hcat_ref[...], w1_ref[...], preferred_element_type=jnp.float32,
                precision=lax.Precision.HIGHEST) + b1_ref[...]
    h = jnp.maximum(h, 0.0)
    out_ref[...] = jnp.dot(h, w2_ref[...], preferred_element_type=jnp.float32,
                           precision=lax.Precision.HIGHEST) + b2_ref[...]


_tc_head = pl.pallas_call(
    _tc_head_body,
    out_shape=jax.ShapeDtypeStruct((G, F), jnp.float32),
)


def kernel(x, edge_index, batch, params):
    src = edge_index[0].reshape(NW, NWIN, W)
    dst = edge_index[1].reshape(NW, NWIN, W)
    zeros = jnp.zeros((RPS, D), jnp.float32)
    batch2 = batch.reshape(1, N)
    h = x
    pooled = []
    for name in ("conv1", "conv2", "conv3", "conv4", "conv5"):
        p = params[name]
        agg = _sc_agg(h, src, dst, zeros)
        h, pool = _tc_layer(
            h, agg[0, :N], agg[1, :N],
            p["W1"], p["b1"].reshape(1, H), p["g1"].reshape(1, H),
            p["beta1"].reshape(1, H), p["W2"], p["b2"].reshape(1, H),
            batch2)
        pooled.append(pool)
    hcat = jnp.concatenate(pooled, axis=1)
    out = _tc_head(hcat, params["lin1_W"], params["lin1_b"].reshape(1, 5 * H),
                   params["lin2_W"], params["lin2_b"].reshape(1, F))
    return out.reshape(-1)
